# baseline (speedup 1.0000x reference)
import functools

import jax
import jax.numpy as jnp
from jax import lax
from jax.experimental import pallas as pl
from jax.experimental.pallas import tpu as pltpu
from jax.experimental.pallas import tpu_sc as plsc

_BM = 512
_FB = 256
_SCH = 32
_GPAD = 128


def _router_kernel(h_ref, gw_ref, p0_ref, p1_ref, g1_ref, g2_ref,
                   eid_ref, nblk_ref, *, bm, nblk_max):
    h = h_ref[...]
    bt = h.shape[0]
    num_e = gw_ref.shape[0]
    logits = jax.lax.dot_general(h, gw_ref[...], (((1,), (1,)), ((), ())),
                                 preferred_element_type=jnp.float32)
    logits = logits - jnp.max(logits, axis=1, keepdims=True)
    ex = jnp.exp(logits)
    probs = ex / jnp.sum(ex, axis=1, keepdims=True)
    iota = jax.lax.broadcasted_iota(jnp.int32, probs.shape, 1)
    m1 = jnp.max(probs, axis=1, keepdims=True)
    i1 = jnp.min(jnp.where(probs == m1, iota, num_e), axis=1, keepdims=True)
    probs2 = jnp.where(iota == i1, -jnp.inf, probs)
    m2 = jnp.max(probs2, axis=1, keepdims=True)
    i2 = jnp.min(jnp.where(probs2 == m2, iota, num_e), axis=1, keepdims=True)
    denom = jnp.maximum(m1 + m2, 1e-9)
    g1_ref[...] = m1 / denom
    g2_ref[...] = m2 / denom

    oh0 = (iota == i1).astype(jnp.float32)
    oh1 = (iota == i2).astype(jnp.float32)
    oh = jnp.concatenate([oh0, oh1], axis=1).astype(jnp.bfloat16)
    ch = min(512, bt)
    ltri = (jax.lax.broadcasted_iota(jnp.int32, (ch, ch), 1)
            < jax.lax.broadcasted_iota(jnp.int32, (ch, ch), 0)
            ).astype(jnp.bfloat16)
    parts = []
    off = jnp.zeros((1, 2 * num_e), jnp.float32)
    for b in range(bt // ch):
        ohb = oh[b * ch:(b + 1) * ch, :]
        rin = jax.lax.dot_general(ltri, ohb, (((1,), (0,)), ((), ())),
                                  preferred_element_type=jnp.float32)
        parts.append(rin + off)
        off = off + jnp.sum(ohb.astype(jnp.float32), axis=0, keepdims=True)
    r = jnp.concatenate(parts, axis=0)
    r0 = r[:, :num_e]
    r1 = r[:, num_e:]
    cnt0 = off[:, :num_e]
    cnt1 = off[:, num_e:]
    count = cnt0 + cnt1
    nblk_e = jnp.floor((count + (bm - 1)) / bm)
    tri = (jax.lax.broadcasted_iota(jnp.int32, (num_e, num_e), 0)
           <= jax.lax.broadcasted_iota(jnp.int32, (num_e, num_e), 1)
           ).astype(jnp.float32)
    cum = jax.lax.dot_general(nblk_e, tri, (((1,), (0,)), ((), ())),
                              preferred_element_type=jnp.float32)
    pad_off = (cum - nblk_e) * bm
    p0 = jnp.sum(oh0 * (pad_off + r0), axis=1, keepdims=True)
    p1 = jnp.sum(oh1 * (pad_off + cnt0 + r1), axis=1, keepdims=True)
    p0_ref[...] = p0.astype(jnp.int32)
    p1_ref[...] = p1.astype(jnp.int32)

    m_iota = jax.lax.broadcasted_iota(jnp.int32, (nblk_max, num_e), 0)
    cum_b = jnp.broadcast_to(cum, (nblk_max, num_e))
    eid = jnp.sum((cum_b <= m_iota.astype(jnp.float32)).astype(jnp.int32),
                  axis=1, keepdims=True)
    eid_ref[...] = jnp.minimum(eid, num_e - 1)
    nblk_ref[...] = cum[:, num_e - 1:num_e].astype(jnp.int32)


def _make_scatter(nw, rows_per_w, hx, npad):
    mesh = plsc.VectorSubcoreMesh(core_axis_name="c", subcore_axis_name="s")
    sch = 16
    nch = rows_per_w // sch

    @functools.partial(
        pl.kernel, mesh=mesh,
        out_type=jax.ShapeDtypeStruct((npad, hx), jnp.float32),
        scratch_types=[
            [pltpu.VMEM((sch,), jnp.int32) for _ in range(4)],
            [pltpu.VMEM((sch, hx), jnp.float32) for _ in range(4)],
            [pltpu.SemaphoreType.DMA for _ in range(4)],
        ])
    def _scatter(hx0_hbm, hx1_hbm, p0_hbm, p1_hbm, out_hbm,
                 idx_v, rows_v, sems):
        wid = lax.axis_index("s") * 2 + lax.axis_index("c")
        base = wid * rows_per_w
        pend = [None, None]
        for c in range(nch):
            s = (c % 2) * 2
            if pend[c % 2] is not None:
                pend[c % 2][0].wait()
                pend[c % 2][1].wait()
            off = base + c * sch
            pltpu.sync_copy(p0_hbm.at[pl.ds(off, sch)], idx_v[s])
            pltpu.sync_copy(p1_hbm.at[pl.ds(off, sch)], idx_v[s + 1])
            pltpu.sync_copy(hx0_hbm.at[pl.ds(off, sch)], rows_v[s])
            pltpu.sync_copy(hx1_hbm.at[pl.ds(off, sch)], rows_v[s + 1])
            cp0 = pltpu.async_copy(rows_v[s], out_hbm.at[idx_v[s]], sems[s])
            cp1 = pltpu.async_copy(rows_v[s + 1], out_hbm.at[idx_v[s + 1]],
                                   sems[s + 1])
            pend[c % 2] = (cp0, cp1)
        for p in pend:
            if p is not None:
                p[0].wait()
                p[1].wait()

    return _scatter


def _make_combine(nw, rows_per_w, h):
    mesh = plsc.VectorSubcoreMesh(core_axis_name="c", subcore_axis_name="s")
    bt = nw * rows_per_w
    cch = 16
    nch = rows_per_w // cch

    @functools.partial(
        pl.kernel, mesh=mesh,
        out_type=jax.ShapeDtypeStruct((bt, h), jnp.float32),
        scratch_types=[
            pltpu.VMEM((rows_per_w,), jnp.int32),
            pltpu.VMEM((rows_per_w,), jnp.int32),
            [pltpu.VMEM((cch, h), jnp.float32) for _ in range(4)],
            [pltpu.SemaphoreType.DMA for _ in range(4)],
        ])
    def _combine(y_hbm, p0_hbm, p1_hbm, out_hbm, idx0_v, idx1_v, r_v, sems):
        wid = lax.axis_index("s") * 2 + lax.axis_index("c")
        base = wid * rows_per_w
        pltpu.sync_copy(p0_hbm.at[pl.ds(base, rows_per_w)], idx0_v)
        pltpu.sync_copy(p1_hbm.at[pl.ds(base, rows_per_w)], idx1_v)

        def start(c):
            s = (c % 2) * 2
            sl = pl.ds(c * cch, cch)
            cp0 = pltpu.async_copy(y_hbm.at[idx0_v.at[sl]], r_v[s], sems[s])
            cp1 = pltpu.async_copy(y_hbm.at[idx1_v.at[sl]], r_v[s + 1],
                                   sems[s + 1])
            return cp0, cp1

        pend = start(0)
        for c in range(nch):
            s = (c % 2) * 2
            pend[0].wait()
            pend[1].wait()
            if c + 1 < nch:
                nxt = start(c + 1)
            else:
                nxt = None

            def row_body(i, _):
                def lane_body(j, _):
                    sl = pl.ds(j * 16, 16)
                    r_v[s][i, sl] = r_v[s][i, sl] + r_v[s + 1][i, sl]
                    return 0
                lax.fori_loop(0, h // 16, lane_body, 0, unroll=8)
                return 0
            lax.fori_loop(0, cch, row_body, 0)
            pltpu.sync_copy(r_v[s],
                            out_hbm.at[pl.ds(base + c * cch, cch)])
            if nxt is not None:
                pend = nxt

    return _combine


def _gmm_kernel(eid_ref, nblk_ref, a_ref, wg_ref, wu_ref, wd_ref, y_ref,
                abf_scr, gate_scr, act_scr, *, h, n_f, fb):
    m = pl.program_id(0)
    f = pl.program_id(1)
    active = m < nblk_ref[0]

    @pl.when(active & (f == 0))
    def _load_a():
        abf_scr[...] = a_ref[:, :h].astype(jnp.bfloat16)
        gate_scr[...] = a_ref[:, h:h + 1]

    @pl.when(active)
    def _compute():
        abf = abf_scr[...]
        gate = gate_scr[...]
        wg = wg_ref[0].astype(jnp.bfloat16)
        wu = wu_ref[0].astype(jnp.bfloat16)
        g = jax.lax.dot_general(abf, wg, (((1,), (1,)), ((), ())),
                                preferred_element_type=jnp.float32)
        u = jax.lax.dot_general(abf, wu, (((1,), (1,)), ((), ())),
                                preferred_element_type=jnp.float32)
        act = g * jax.nn.sigmoid(g) * u * gate
        act_scr[:, pl.ds(f * fb, fb)] = act.astype(jnp.bfloat16)

        @pl.when(f == n_f - 1)
        def _down():
            wd = wd_ref[0].astype(jnp.bfloat16)
            y_ref[...] = jax.lax.dot_general(
                act_scr[...], wd, (((1,), (1,)), ((), ())),
                preferred_element_type=jnp.float32)


def kernel(hidden_states, gate_w, gate_proj_w, up_proj_w, down_proj_w):
    orig_shape = hidden_states.shape
    H = orig_shape[-1]
    h = hidden_states.reshape(-1, H)
    BT = h.shape[0]
    E, F, _ = gate_proj_w.shape
    K = 2
    nblk_max = (BT * K) // _BM + E
    npad = nblk_max * _BM
    n_f = F // _FB
    HX = H + _GPAD
    NW = 32
    rows_per_w = BT // NW

    p0, p1, g1, g2, eid, nblk = pl.pallas_call(
        functools.partial(_router_kernel, bm=_BM, nblk_max=nblk_max),
        out_shape=[
            jax.ShapeDtypeStruct((BT, 1), jnp.int32),
            jax.ShapeDtypeStruct((BT, 1), jnp.int32),
            jax.ShapeDtypeStruct((BT, 1), jnp.float32),
            jax.ShapeDtypeStruct((BT, 1), jnp.float32),
            jax.ShapeDtypeStruct((nblk_max, 1), jnp.int32),
            jax.ShapeDtypeStruct((1, 1), jnp.int32),
        ],
    )(h, gate_w)

    p0v = p0.reshape(BT)
    p1v = p1.reshape(BT)
    eidv = eid.reshape(nblk_max)
    nblkv = nblk.reshape(1)

    hx0 = jnp.concatenate([h, jnp.broadcast_to(g1, (BT, _GPAD))], axis=1)
    hx1 = jnp.concatenate([h, jnp.broadcast_to(g2, (BT, _GPAD))], axis=1)

    sorted_hx = _make_scatter(NW, rows_per_w, HX, npad)(hx0, hx1, p0v, p1v)

    grid_spec = pltpu.PrefetchScalarGridSpec(
        num_scalar_prefetch=2,
        grid=(nblk_max, n_f),
        in_specs=[
            pl.BlockSpec(
                (_BM, HX),
                lambda m, f, eid, nblk: (
                    jnp.where(m < nblk[0], m, jnp.maximum(nblk[0] - 1, 0)),
                    0)),
            pl.BlockSpec(
                (1, _FB, H),
                lambda m, f, eid, nblk: (
                    eid[m], jnp.where(m < nblk[0], f, 0), 0)),
            pl.BlockSpec(
                (1, _FB, H),
                lambda m, f, eid, nblk: (
                    eid[m], jnp.where(m < nblk[0], f, 0), 0)),
            pl.BlockSpec(
                (1, H, F),
                lambda m, f, eid, nblk: (eid[m], 0, 0)),
        ],
        out_specs=pl.BlockSpec((_BM, H), lambda m, f, eid, nblk: (m, 0)),
        scratch_shapes=[
            pltpu.VMEM((_BM, H), jnp.bfloat16),
            pltpu.VMEM((_BM, 1), jnp.float32),
            pltpu.VMEM((_BM, F), jnp.bfloat16),
        ],
    )
    y_sorted = pl.pallas_call(
        functools.partial(_gmm_kernel, h=H, n_f=n_f, fb=_FB),
        grid_spec=grid_spec,
        out_shape=jax.ShapeDtypeStruct((npad, H), jnp.float32),
        compiler_params=pltpu.CompilerParams(
            dimension_semantics=("arbitrary", "arbitrary"),
        ),
    )(eidv, nblkv, sorted_hx, gate_proj_w, up_proj_w, down_proj_w)

    out = _make_combine(NW, rows_per_w, H)(y_sorted, p0v, p1v)
    return out.reshape(orig_shape)


# speedup vs baseline: 1.2143x; 1.2143x over previous
import functools

import jax
import jax.numpy as jnp
from jax import lax
from jax.experimental import pallas as pl
from jax.experimental.pallas import tpu as pltpu
from jax.experimental.pallas import tpu_sc as plsc

_BM = 1024
_FB = 256
_SCH = 32
_GPAD = 128


def _router_kernel(h_ref, gw_ref, p0_ref, p1_ref, g1_ref, g2_ref,
                   eid_ref, nblk_ref, *, bm, nblk_max):
    h = h_ref[...]
    bt = h.shape[0]
    num_e = gw_ref.shape[0]
    logits = jax.lax.dot_general(h, gw_ref[...], (((1,), (1,)), ((), ())),
                                 preferred_element_type=jnp.float32)
    logits = logits - jnp.max(logits, axis=1, keepdims=True)
    ex = jnp.exp(logits)
    probs = ex / jnp.sum(ex, axis=1, keepdims=True)
    iota = jax.lax.broadcasted_iota(jnp.int32, probs.shape, 1)
    m1 = jnp.max(probs, axis=1, keepdims=True)
    i1 = jnp.min(jnp.where(probs == m1, iota, num_e), axis=1, keepdims=True)
    probs2 = jnp.where(iota == i1, -jnp.inf, probs)
    m2 = jnp.max(probs2, axis=1, keepdims=True)
    i2 = jnp.min(jnp.where(probs2 == m2, iota, num_e), axis=1, keepdims=True)
    denom = jnp.maximum(m1 + m2, 1e-9)
    gw_lanes = g1_ref.shape[1]
    g1_ref[...] = jnp.broadcast_to(m1 / denom, (bt, gw_lanes))
    g2_ref[...] = jnp.broadcast_to(m2 / denom, (bt, gw_lanes))

    oh0 = (iota == i1).astype(jnp.float32)
    oh1 = (iota == i2).astype(jnp.float32)
    oh = jnp.concatenate([oh0, oh1], axis=1).astype(jnp.bfloat16)
    ch = min(512, bt)
    ltri = (jax.lax.broadcasted_iota(jnp.int32, (ch, ch), 1)
            < jax.lax.broadcasted_iota(jnp.int32, (ch, ch), 0)
            ).astype(jnp.bfloat16)
    parts = []
    off = jnp.zeros((1, 2 * num_e), jnp.float32)
    for b in range(bt // ch):
        ohb = oh[b * ch:(b + 1) * ch, :]
        rin = jax.lax.dot_general(ltri, ohb, (((1,), (0,)), ((), ())),
                                  preferred_element_type=jnp.float32)
        parts.append(rin + off)
        off = off + jnp.sum(ohb.astype(jnp.float32), axis=0, keepdims=True)
    r = jnp.concatenate(parts, axis=0)
    r0 = r[:, :num_e]
    r1 = r[:, num_e:]
    cnt0 = off[:, :num_e]
    cnt1 = off[:, num_e:]
    count = cnt0 + cnt1
    nblk_e = jnp.floor((count + (bm - 1)) / bm)
    tri = (jax.lax.broadcasted_iota(jnp.int32, (num_e, num_e), 0)
           <= jax.lax.broadcasted_iota(jnp.int32, (num_e, num_e), 1)
           ).astype(jnp.float32)
    cum = jax.lax.dot_general(nblk_e, tri, (((1,), (0,)), ((), ())),
                              preferred_element_type=jnp.float32)
    pad_off = (cum - nblk_e) * bm
    p0 = jnp.sum(oh0 * (pad_off + r0), axis=1, keepdims=True)
    p1 = jnp.sum(oh1 * (pad_off + cnt0 + r1), axis=1, keepdims=True)
    p0_ref[...] = p0.astype(jnp.int32)
    p1_ref[...] = p1.astype(jnp.int32)

    m_iota = jax.lax.broadcasted_iota(jnp.int32, (nblk_max, num_e), 0)
    cum_b = jnp.broadcast_to(cum, (nblk_max, num_e))
    eid = jnp.sum((cum_b <= m_iota.astype(jnp.float32)).astype(jnp.int32),
                  axis=1, keepdims=True)
    eid_ref[...] = jnp.minimum(eid, num_e - 1)
    nblk_ref[...] = cum[:, num_e - 1:num_e].astype(jnp.int32)


def _make_scatter(nw, rows_per_w, h, npad, gw_lanes):
    mesh = plsc.VectorSubcoreMesh(core_axis_name="c", subcore_axis_name="s")
    sch = 16
    nch = rows_per_w // sch

    @functools.partial(
        pl.kernel, mesh=mesh,
        out_type=(jax.ShapeDtypeStruct((npad, h), jnp.float32),
                  jax.ShapeDtypeStruct((npad, gw_lanes), jnp.float32)),
        scratch_types=[
            [pltpu.VMEM((sch,), jnp.int32) for _ in range(4)],
            [pltpu.VMEM((sch, h), jnp.float32) for _ in range(2)],
            [pltpu.VMEM((sch, gw_lanes), jnp.float32) for _ in range(4)],
            [pltpu.SemaphoreType.DMA for _ in range(6)],
        ])
    def _scatter(h_hbm, g1_hbm, g2_hbm, p0_hbm, p1_hbm, out_hbm, gout_hbm,
                 idx_v, rows_v, gv, sems):
        wid = lax.axis_index("s") * 2 + lax.axis_index("c")
        base = wid * rows_per_w
        pend = [None, None]
        for c in range(nch):
            b = c % 2
            s = b * 2
            if pend[b] is not None:
                for cp in pend[b]:
                    cp.wait()
            off = base + c * sch
            pltpu.sync_copy(p0_hbm.at[pl.ds(off, sch)], idx_v[s])
            pltpu.sync_copy(p1_hbm.at[pl.ds(off, sch)], idx_v[s + 1])
            pltpu.sync_copy(h_hbm.at[pl.ds(off, sch)], rows_v[b])
            pltpu.sync_copy(g1_hbm.at[pl.ds(off, sch)], gv[s])
            pltpu.sync_copy(g2_hbm.at[pl.ds(off, sch)], gv[s + 1])
            pend[b] = (
                pltpu.async_copy(rows_v[b], out_hbm.at[idx_v[s]],
                                 sems[s]),
                pltpu.async_copy(rows_v[b], out_hbm.at[idx_v[s + 1]],
                                 sems[s + 1]),
                pltpu.async_copy(gv[s], gout_hbm.at[idx_v[s]],
                                 sems[4 + b]),
                pltpu.async_copy(gv[s + 1], gout_hbm.at[idx_v[s + 1]],
                                 sems[4 + b]),
            )
        for p in pend:
            if p is not None:
                for cp in p:
                    cp.wait()

    return _scatter


def _make_combine(nw, rows_per_w, h):
    mesh = plsc.VectorSubcoreMesh(core_axis_name="c", subcore_axis_name="s")
    bt = nw * rows_per_w
    cch = 16
    nch = rows_per_w // cch

    @functools.partial(
        pl.kernel, mesh=mesh,
        out_type=jax.ShapeDtypeStruct((bt, h), jnp.float32),
        scratch_types=[
            pltpu.VMEM((rows_per_w,), jnp.int32),
            pltpu.VMEM((rows_per_w,), jnp.int32),
            [pltpu.VMEM((cch, h), jnp.float32) for _ in range(4)],
            [pltpu.SemaphoreType.DMA for _ in range(4)],
        ])
    def _combine(y_hbm, p0_hbm, p1_hbm, out_hbm, idx0_v, idx1_v, r_v, sems):
        wid = lax.axis_index("s") * 2 + lax.axis_index("c")
        base = wid * rows_per_w
        pltpu.sync_copy(p0_hbm.at[pl.ds(base, rows_per_w)], idx0_v)
        pltpu.sync_copy(p1_hbm.at[pl.ds(base, rows_per_w)], idx1_v)

        def start(c):
            s = (c % 2) * 2
            sl = pl.ds(c * cch, cch)
            cp0 = pltpu.async_copy(y_hbm.at[idx0_v.at[sl]], r_v[s], sems[s])
            cp1 = pltpu.async_copy(y_hbm.at[idx1_v.at[sl]], r_v[s + 1],
                                   sems[s + 1])
            return cp0, cp1

        pend = start(0)
        for c in range(nch):
            s = (c % 2) * 2
            pend[0].wait()
            pend[1].wait()
            if c + 1 < nch:
                nxt = start(c + 1)
            else:
                nxt = None

            def row_body(i, _):
                def lane_body(j, _):
                    sl = pl.ds(j * 16, 16)
                    r_v[s][i, sl] = r_v[s][i, sl] + r_v[s + 1][i, sl]
                    return 0
                lax.fori_loop(0, h // 16, lane_body, 0, unroll=8)
                return 0
            lax.fori_loop(0, cch, row_body, 0)
            pltpu.sync_copy(r_v[s],
                            out_hbm.at[pl.ds(base + c * cch, cch)])
            if nxt is not None:
                pend = nxt

    return _combine


def _gmm_kernel(eid_ref, nblk_ref, a_ref, gs_ref, wg_ref, wu_ref, wd_ref,
                y_ref, abf_scr, act_scr, *, h, n_f, fb):
    m = pl.program_id(0)
    f = pl.program_id(1)
    active = m < nblk_ref[0]

    @pl.when(active & (f == 0))
    def _load_a():
        abf_scr[...] = a_ref[...].astype(jnp.bfloat16)

    @pl.when(active)
    def _compute():
        abf = abf_scr[...]
        gate = gs_ref[:, :1]
        wg = wg_ref[0].astype(jnp.bfloat16)
        wu = wu_ref[0].astype(jnp.bfloat16)
        g = jax.lax.dot_general(abf, wg, (((1,), (1,)), ((), ())),
                                preferred_element_type=jnp.float32)
        u = jax.lax.dot_general(abf, wu, (((1,), (1,)), ((), ())),
                                preferred_element_type=jnp.float32)
        act = g * jax.nn.sigmoid(g) * u * gate
        act_scr[:, pl.ds(f * fb, fb)] = act.astype(jnp.bfloat16)

        @pl.when(f == n_f - 1)
        def _down():
            wd = wd_ref[0].astype(jnp.bfloat16)
            y_ref[...] = jax.lax.dot_general(
                act_scr[...], wd, (((1,), (1,)), ((), ())),
                preferred_element_type=jnp.float32)


def kernel(hidden_states, gate_w, gate_proj_w, up_proj_w, down_proj_w):
    orig_shape = hidden_states.shape
    H = orig_shape[-1]
    h = hidden_states.reshape(-1, H)
    BT = h.shape[0]
    E, F, _ = gate_proj_w.shape
    K = 2
    nblk_max = (BT * K) // _BM + E
    npad = nblk_max * _BM
    n_f = F // _FB
    NW = 32
    rows_per_w = BT // NW

    p0, p1, g1, g2, eid, nblk = pl.pallas_call(
        functools.partial(_router_kernel, bm=_BM, nblk_max=nblk_max),
        out_shape=[
            jax.ShapeDtypeStruct((BT, 1), jnp.int32),
            jax.ShapeDtypeStruct((BT, 1), jnp.int32),
            jax.ShapeDtypeStruct((BT, _GPAD), jnp.float32),
            jax.ShapeDtypeStruct((BT, _GPAD), jnp.float32),
            jax.ShapeDtypeStruct((nblk_max, 1), jnp.int32),
            jax.ShapeDtypeStruct((1, 1), jnp.int32),
        ],
    )(h, gate_w)

    p0v = p0.reshape(BT)
    p1v = p1.reshape(BT)
    eidv = eid.reshape(nblk_max)
    nblkv = nblk.reshape(1)

    sorted_h, sorted_g = _make_scatter(NW, rows_per_w, H, npad, _GPAD)(
        h, g1, g2, p0v, p1v)

    grid_spec = pltpu.PrefetchScalarGridSpec(
        num_scalar_prefetch=2,
        grid=(nblk_max, n_f),
        in_specs=[
            pl.BlockSpec(
                (_BM, H),
                lambda m, f, eid, nblk: (
                    jnp.where(m < nblk[0], m, jnp.maximum(nblk[0] - 1, 0)),
                    0)),
            pl.BlockSpec(
                (_BM, _GPAD),
                lambda m, f, eid, nblk: (
                    jnp.where(m < nblk[0], m, jnp.maximum(nblk[0] - 1, 0)),
                    0)),
            pl.BlockSpec(
                (1, _FB, H),
                lambda m, f, eid, nblk: (
                    eid[m], jnp.where(m < nblk[0], f, 0), 0)),
            pl.BlockSpec(
                (1, _FB, H),
                lambda m, f, eid, nblk: (
                    eid[m], jnp.where(m < nblk[0], f, 0), 0)),
            pl.BlockSpec(
                (1, H, F),
                lambda m, f, eid, nblk: (eid[m], 0, 0)),
        ],
        out_specs=pl.BlockSpec((_BM, H), lambda m, f, eid, nblk: (m, 0)),
        scratch_shapes=[
            pltpu.VMEM((_BM, H), jnp.bfloat16),
            pltpu.VMEM((_BM, F), jnp.bfloat16),
        ],
    )
    y_sorted = pl.pallas_call(
        functools.partial(_gmm_kernel, h=H, n_f=n_f, fb=_FB),
        grid_spec=grid_spec,
        out_shape=jax.ShapeDtypeStruct((npad, H), jnp.float32),
        compiler_params=pltpu.CompilerParams(
            dimension_semantics=("arbitrary", "arbitrary"),
        ),
    )(eidv, nblkv, sorted_h, sorted_g, gate_proj_w, up_proj_w,
      down_proj_w)

    out = _make_combine(NW, rows_per_w, H)(y_sorted, p0v, p1v)
    return out.reshape(orig_shape)
